# baseline (speedup 1.0000x reference)
import functools

import jax
import jax.numpy as jnp
from jax import lax
from jax.experimental import pallas as pl
from jax.experimental.pallas import tpu as pltpu

try:
    from jax.experimental.pallas import tpu_sc as plsc
except ImportError:
    plsc = None

TOP_K = 8
N_EXPERTS = 64
N_GROUP = 8
GROUP_SIZE = N_EXPERTS // N_GROUP
TOPK_GROUP = 4
ROUTE_SCALE = 2.5
DIM = 2048
TOKENS = 16384

BLOCK_T = 512
_NEG_INF = float("-inf")


def _score_body(x_ref, wt_ref, bias_ref, s_ref, raw_ref):
    x = x_ref[...]
    wt = wt_ref[...]
    logits = jax.lax.dot_general(
        x, wt, (((1,), (0,)), ((), ())), preferred_element_type=jnp.float32
    )
    scores = jax.nn.sigmoid(logits.T)
    raw_ref[...] = scores
    s_ref[...] = scores + bias_ref[...]


def _scores_tc(x, wt, bias):
    n_tokens = x.shape[0]
    grid = (n_tokens // BLOCK_T,)
    return pl.pallas_call(
        _score_body,
        grid=grid,
        in_specs=[
            pl.BlockSpec((BLOCK_T, DIM), lambda i: (i, 0)),
            pl.BlockSpec((DIM, N_EXPERTS), lambda i: (0, 0)),
            pl.BlockSpec((N_EXPERTS, 1), lambda i: (0, 0)),
        ],
        out_specs=(
            pl.BlockSpec((N_EXPERTS, BLOCK_T), lambda i: (0, i)),
            pl.BlockSpec((N_EXPERTS, BLOCK_T), lambda i: (0, i)),
        ),
        out_shape=(
            jax.ShapeDtypeStruct((N_EXPERTS, n_tokens), jnp.float32),
            jax.ShapeDtypeStruct((N_EXPERTS, n_tokens), jnp.float32),
        ),
    )(x, wt, bias)


def _route_sc(s_t, raw_t):
    info = plsc.get_sparse_core_info()
    nc, ns, lanes = info.num_cores, info.num_subcores, info.num_lanes
    nw = nc * ns
    n_tokens = s_t.shape[1]
    tpw = n_tokens // nw
    n_chunks = tpw // lanes

    mesh = plsc.VectorSubcoreMesh(core_axis_name="c", subcore_axis_name="s")

    @functools.partial(
        pl.kernel,
        mesh=mesh,
        compiler_params=pltpu.CompilerParams(use_tc_tiling_on_sc=False, needs_layout_passes=False),
        out_type=(
            jax.ShapeDtypeStruct((TOP_K, n_tokens), jnp.int32),
            jax.ShapeDtypeStruct((TOP_K, n_tokens), jnp.float32),
        ),
        scratch_types=[
            pltpu.VMEM((N_EXPERTS, tpw), jnp.float32),
            pltpu.VMEM((N_EXPERTS, tpw), jnp.float32),
            pltpu.VMEM((TOP_K, tpw), jnp.int32),
            pltpu.VMEM((TOP_K, tpw), jnp.float32),
        ],
    )
    def route(s_hbm, raw_hbm, oi_hbm, ow_hbm, sbuf, rbuf, oibuf, owbuf):
        wid = lax.axis_index("s") * nc + lax.axis_index("c")
        t0 = wid * tpw
        pltpu.sync_copy(s_hbm.at[:, pl.ds(t0, tpw)], sbuf)
        pltpu.sync_copy(raw_hbm.at[:, pl.ds(t0, tpw)], rbuf)

        lane = lax.iota(jnp.int32, lanes)

        def chunk_body(c, _):
            base = c * lanes
            col = base + lane

            gsum = []
            gmax1 = []
            for g in range(N_GROUP):
                m1 = sbuf[g * GROUP_SIZE, pl.ds(base, lanes)]
                m2 = jnp.full((lanes,), _NEG_INF, dtype=jnp.float32)
                for j in range(1, GROUP_SIZE):
                    x = sbuf[g * GROUP_SIZE + j, pl.ds(base, lanes)]
                    hi = jnp.maximum(m1, x)
                    lo = jnp.minimum(m1, x)
                    m2 = jnp.maximum(m2, lo)
                    m1 = hi
                gsum.append(m1 + m2)
                gmax1.append(m1)

            keep = []
            for i in range(N_GROUP):
                rank = jnp.zeros((lanes,), jnp.int32)
                for j in range(N_GROUP):
                    if j == i:
                        continue
                    if j < i:
                        beats = gsum[j] >= gsum[i]
                    else:
                        beats = gsum[j] > gsum[i]
                    rank = rank + jnp.where(beats, 1, 0)
                keep.append(rank < TOPK_GROUP)

            gmax = []
            for g in range(N_GROUP):
                kg = keep[g]
                for j in range(GROUP_SIZE):
                    x = sbuf[g * GROUP_SIZE + j, pl.ds(base, lanes)]
                    sbuf[g * GROUP_SIZE + j, pl.ds(base, lanes)] = jnp.where(
                        kg, x, 0.0
                    )
                gmax.append(jnp.where(kg, gmax1[g], 0.0))

            wsum = jnp.zeros((lanes,), jnp.float32)
            widx = []
            wval = []
            for k in range(TOP_K):
                m = gmax[0]
                for g in range(1, N_GROUP):
                    m = jnp.maximum(m, gmax[g])
                gid = jnp.full((lanes,), N_GROUP, jnp.int32)
                for g in range(N_GROUP - 1, -1, -1):
                    gid = jnp.where(gmax[g] == m, g, gid)
                xs = []
                for j in range(GROUP_SIZE):
                    row = gid * GROUP_SIZE + j
                    xs.append(plsc.load_gather(sbuf, [row, col]))
                jstar = jnp.full((lanes,), GROUP_SIZE, jnp.int32)
                for j in range(GROUP_SIZE - 1, -1, -1):
                    jstar = jnp.where(xs[j] == m, j, jstar)
                estar = gid * GROUP_SIZE + jstar
                wv = plsc.load_gather(rbuf, [estar, col])
                plsc.store_scatter(
                    sbuf, [estar, col], jnp.full((lanes,), _NEG_INF, jnp.float32)
                )
                newmax = jnp.full((lanes,), _NEG_INF, jnp.float32)
                for j in range(GROUP_SIZE):
                    newmax = jnp.maximum(
                        newmax, jnp.where(jstar == j, _NEG_INF, xs[j])
                    )
                for g in range(N_GROUP):
                    gmax[g] = jnp.where(gid == g, newmax, gmax[g])
                widx.append(estar)
                wval.append(wv)
                wsum = wsum + wv

            scale = ROUTE_SCALE / (wsum + 1e-20)
            for k in range(TOP_K):
                oibuf[k, pl.ds(base, lanes)] = widx[k]
                owbuf[k, pl.ds(base, lanes)] = wval[k] * scale
            return _

        lax.fori_loop(0, n_chunks, chunk_body, 0)

        pltpu.sync_copy(oibuf, oi_hbm.at[:, pl.ds(t0, tpw)])
        pltpu.sync_copy(owbuf, ow_hbm.at[:, pl.ds(t0, tpw)])

    return route(s_t, raw_t)


def kernel(hidden_states, weight, e_score_correction_bias, interpret=False):
    x = hidden_states.reshape(-1, DIM).astype(jnp.float32)
    wt = weight.astype(jnp.float32).T
    bias = e_score_correction_bias.astype(jnp.float32).reshape(N_EXPERTS, 1)
    s_t, raw_t = _scores_tc(x, wt, bias)
    oi_t, ow_t = _route_sc(s_t, raw_t)
    return oi_t.T, ow_t.T


# speedup vs baseline: 1.1677x; 1.1677x over previous
import functools

import jax
import jax.numpy as jnp
from jax import lax
from jax.experimental import pallas as pl
from jax.experimental.pallas import tpu as pltpu
from jax.experimental.pallas import tpu_sc as plsc

TOP_K = 8
N_EXPERTS = 64
N_GROUP = 8
GROUP_SIZE = N_EXPERTS // N_GROUP
TOPK_GROUP = 4
ROUTE_SCALE = 2.5
DIM = 2048

BLOCK_T = 2048
_NEG_INF = float("-inf")


def _score_body(x_ref, wt_ref, bias_ref, s_ref):
    x = x_ref[...]
    wt = wt_ref[...]
    logits = jax.lax.dot_general(
        x, wt, (((1,), (0,)), ((), ())), preferred_element_type=jnp.float32
    )
    s_ref[...] = jax.nn.sigmoid(logits.T) + bias_ref[...]


def _scores_tc(x, wt, bias):
    n_tokens = x.shape[0]
    grid = (n_tokens // BLOCK_T,)
    return pl.pallas_call(
        _score_body,
        grid=grid,
        in_specs=[
            pl.BlockSpec((BLOCK_T, DIM), lambda i: (i, 0)),
            pl.BlockSpec((DIM, N_EXPERTS), lambda i: (0, 0)),
            pl.BlockSpec((N_EXPERTS, 1), lambda i: (0, 0)),
        ],
        out_specs=pl.BlockSpec((N_EXPERTS, BLOCK_T), lambda i: (0, i)),
        out_shape=jax.ShapeDtypeStruct((N_EXPERTS, n_tokens), jnp.float32),
    )(x, wt, bias)


def _route_sc(s_t):
    info = plsc.get_sparse_core_info()
    nc, ns, lanes = info.num_cores, info.num_subcores, info.num_lanes
    nw = nc * ns
    n_tokens = s_t.shape[1]
    tpw = n_tokens // nw
    n_chunks = tpw // lanes

    mesh = plsc.VectorSubcoreMesh(core_axis_name="c", subcore_axis_name="s")

    @functools.partial(
        pl.kernel,
        mesh=mesh,
        compiler_params=pltpu.CompilerParams(
            use_tc_tiling_on_sc=False, needs_layout_passes=False
        ),
        out_type=(
            jax.ShapeDtypeStruct((TOP_K, n_tokens), jnp.int32),
            jax.ShapeDtypeStruct((TOP_K, n_tokens), jnp.float32),
        ),
        scratch_types=[
            pltpu.VMEM((N_EXPERTS, tpw), jnp.float32),
            pltpu.VMEM((N_GROUP, tpw), jnp.float32),
            pltpu.VMEM((TOP_K, tpw), jnp.int32),
            pltpu.VMEM((TOP_K, tpw), jnp.float32),
        ],
    )
    def route(s_hbm, oi_hbm, ow_hbm, sbuf, kbuf, oibuf, owbuf):
        wid = lax.axis_index("s") * nc + lax.axis_index("c")
        t0 = wid * tpw
        pltpu.sync_copy(s_hbm.at[:, pl.ds(t0, tpw)], sbuf)

        lane = lax.iota(jnp.int32, lanes)

        @plsc.parallel_loop(0, n_chunks, 1, unroll=2)
        def chunk_body(c):
            base = c * lanes
            col = base + lane

            gsum = []
            gmax1 = []
            for g in range(N_GROUP):
                m1 = sbuf[g * GROUP_SIZE, pl.ds(base, lanes)]
                m2 = jnp.full((lanes,), _NEG_INF, dtype=jnp.float32)
                for j in range(1, GROUP_SIZE):
                    x = sbuf[g * GROUP_SIZE + j, pl.ds(base, lanes)]
                    hi = jnp.maximum(m1, x)
                    lo = jnp.minimum(m1, x)
                    m2 = jnp.maximum(m2, lo)
                    m1 = hi
                gsum.append(m1 + m2)
                gmax1.append(m1)

            gmax = []
            for i in range(N_GROUP):
                rank = jnp.zeros((lanes,), jnp.int32)
                for j in range(N_GROUP):
                    if j == i:
                        continue
                    if j < i:
                        beats = gsum[j] >= gsum[i]
                    else:
                        beats = gsum[j] > gsum[i]
                    rank = rank + jnp.where(beats, 1, 0)
                kg = rank < TOPK_GROUP
                kbuf[i, pl.ds(base, lanes)] = jnp.where(kg, 1.0, 0.0)
                gmax.append(jnp.where(kg, gmax1[i], 0.0))

            wsum = jnp.zeros((lanes,), jnp.float32)
            widx = []
            wval = []
            for k in range(TOP_K):
                m = gmax[0]
                for g in range(1, N_GROUP):
                    m = jnp.maximum(m, gmax[g])
                gid = jnp.full((lanes,), N_GROUP - 1, jnp.int32)
                for g in range(N_GROUP - 2, -1, -1):
                    gid = jnp.where(gmax[g] == m, g, gid)
                kmask = plsc.load_gather(kbuf, [gid, col]) > 0.5
                xs = []
                for j in range(GROUP_SIZE):
                    row = gid * GROUP_SIZE + j
                    v = plsc.load_gather(sbuf, [row, col])
                    xs.append(jnp.where(kmask, v, jnp.minimum(v, 0.0)))
                jstar = jnp.full((lanes,), GROUP_SIZE - 1, jnp.int32)
                for j in range(GROUP_SIZE - 2, -1, -1):
                    jstar = jnp.where(xs[j] == m, j, jstar)
                estar = gid * GROUP_SIZE + jstar
                wv = plsc.load_gather(sbuf, [estar, col])
                plsc.store_scatter(
                    sbuf, [estar, col], jnp.full((lanes,), _NEG_INF, jnp.float32)
                )
                newmax = jnp.full((lanes,), _NEG_INF, jnp.float32)
                for j in range(GROUP_SIZE):
                    newmax = jnp.maximum(
                        newmax, jnp.where(jstar == j, _NEG_INF, xs[j])
                    )
                for g in range(N_GROUP):
                    gmax[g] = jnp.where(gid == g, newmax, gmax[g])
                widx.append(estar)
                wval.append(wv)
                wsum = wsum + wv

            scale = ROUTE_SCALE / (wsum + 1e-20)
            for k in range(TOP_K):
                oibuf[k, pl.ds(base, lanes)] = widx[k]
                owbuf[k, pl.ds(base, lanes)] = wval[k] * scale

        pltpu.sync_copy(oibuf, oi_hbm.at[:, pl.ds(t0, tpw)])
        pltpu.sync_copy(owbuf, ow_hbm.at[:, pl.ds(t0, tpw)])

    return route(s_t)


def kernel(hidden_states, weight, e_score_correction_bias):
    x = hidden_states.reshape(-1, DIM).astype(jnp.float32)
    wt = weight.astype(jnp.float32).T
    bias = e_score_correction_bias.astype(jnp.float32).reshape(N_EXPERTS, 1)
    s_t = _scores_tc(x, wt, bias)
    oi_t, ow_t = _route_sc(s_t)
    return oi_t.T, ow_t.T
